# baseline (speedup 1.0000x reference)
import functools

import jax
import jax.numpy as jnp
from jax import lax
from jax.experimental import pallas as pl
from jax.experimental.pallas import tpu as pltpu
from jax.experimental.pallas import tpu_sc as plsc

N = 4096
L = 16
NC = 2
NS = 16
NW = NC * NS
ROWS_PER_W = N // NW
RBLK = 4
NBLK = ROWS_PER_W // RBLK
CHUNKS = N // L
NWORDS = N // 2

_mesh = plsc.VectorSubcoreMesh(core_axis_name="c", subcore_axis_name="s")


@functools.partial(
    pl.kernel,
    out_type=jax.ShapeDtypeStruct((N, N), jnp.float32),
    mesh=_mesh,
    scratch_types=[
        pltpu.VMEM((RBLK, N), jnp.float32),
        pltpu.VMEM((RBLK, N), jnp.float32),
        pltpu.VMEM((RBLK, NWORDS), jnp.float32),
        pltpu.VMEM((RBLK, NWORDS), jnp.float32),
        pltpu.VMEM((RBLK, N), jnp.float32),
        pltpu.VMEM((RBLK, N), jnp.float32),
        pltpu.SemaphoreType.DMA,
        pltpu.SemaphoreType.DMA,
        pltpu.SemaphoreType.DMA,
        pltpu.SemaphoreType.DMA,
    ],
    compiler_params=pltpu.CompilerParams(needs_layout_passes=False),
)
def _masked_cumsum_sc(x_hbm, m_hbm, out_hbm, xv0, xv1, mv0, mv1, ov0, ov1,
                      sin0, sin1, sout0, sout1):
    wid = lax.axis_index("s") * NC + lax.axis_index("c")
    row0 = wid * ROWS_PER_W

    def blk_row(b):
        return row0 + jnp.minimum(b, NBLK - 1) * RBLK

    def start_in(b, xv, mv, sem):
        r = blk_row(b)
        pltpu.make_async_copy(x_hbm.at[pl.ds(r, RBLK)], xv, sem).start()
        pltpu.make_async_copy(m_hbm.at[pl.ds(r, RBLK)], mv, sem).start()

    def wait_in(xv, mv, sem):
        pltpu.make_async_copy(x_hbm.at[pl.ds(row0, RBLK)], xv, sem).wait()
        pltpu.make_async_copy(m_hbm.at[pl.ds(row0, RBLK)], mv, sem).wait()

    def start_out(b, ov, sem):
        r = blk_row(b)
        pltpu.make_async_copy(ov, out_hbm.at[pl.ds(r, RBLK)], sem).start()

    def wait_out(ov, sem):
        pltpu.make_async_copy(ov, out_hbm.at[pl.ds(row0, RBLK)], sem).wait()

    def compute_block(xv, mv, ov):
        def pair(i, carries):
            carries = list(carries)
            msl = pl.ds(i * L, L)
            for rr in range(RBLK):
                w = plsc.bitcast(mv[rr, msl], jnp.int32)
                for half in range(2):
                    bits = (w << 16) if half == 0 else (w & jnp.int32(-65536))
                    mf = plsc.bitcast(bits, jnp.float32)
                    sl = pl.ds((2 * i + half) * L, L)
                    masked = xv[rr, sl] * mf
                    s = jnp.cumsum(masked)
                    ov[rr, sl] = s + carries[rr]
                    carries[rr] = s[L - 1] + carries[rr]
            return tuple(carries)

        lax.fori_loop(0, CHUNKS // 2, pair, (jnp.float32(0.0),) * RBLK)

    def do_pair(k, carry_unused):
        b0 = 2 * k
        b1 = 2 * k + 1
        wait_in(xv0, mv0, sin0)

        @pl.when(k > 0)
        def _():
            wait_out(ov0, sout0)

        compute_block(xv0, mv0, ov0)
        start_out(b0, ov0, sout0)
        start_in(b0 + 2, xv0, mv0, sin0)
        wait_in(xv1, mv1, sin1)

        @pl.when(k > 0)
        def _():
            wait_out(ov1, sout1)

        compute_block(xv1, mv1, ov1)
        start_out(b1, ov1, sout1)
        start_in(b1 + 2, xv1, mv1, sin1)
        return carry_unused

    start_in(0, xv0, mv0, sin0)
    start_in(1, xv1, mv1, sin1)
    lax.fori_loop(0, NBLK // 2, do_pair, 0)
    wait_out(ov0, sout0)
    wait_out(ov1, sout1)
    wait_in(xv0, mv0, sin0)
    wait_in(xv1, mv1, sin1)


def kernel(x, mask):
    m32 = mask.astype(jnp.int32).reshape(N, CHUNKS // 2, 2, L)
    mw = m32[:, :, 0, :] * 0x3F80 + m32[:, :, 1, :] * 0x3F800000
    mwf = lax.bitcast_convert_type(mw.reshape(N, NWORDS), jnp.float32)
    return _masked_cumsum_sc(x, mwf)


# speedup vs baseline: 4.3881x; 4.3881x over previous
import functools

import jax
import jax.numpy as jnp
from jax import lax
from jax.experimental import pallas as pl
from jax.experimental.pallas import tpu as pltpu
from jax.experimental.pallas import tpu_sc as plsc

N = 4096
L = 16
NC = 2
NS = 16
NW = NC * NS
ROWS_PER_W = N // NW
RBLK = 4
NBLK = ROWS_PER_W // RBLK
CHUNKS = N // L

_mesh = plsc.VectorSubcoreMesh(core_axis_name="c", subcore_axis_name="s")


@functools.partial(
    pl.kernel,
    out_type=jax.ShapeDtypeStruct((N, N), jnp.float32),
    mesh=_mesh,
    scratch_types=[
        pltpu.VMEM((RBLK, N), jnp.float32),
        pltpu.VMEM((RBLK, N), jnp.float32),
        pltpu.VMEM((RBLK, N), jnp.float32),
        pltpu.VMEM((RBLK, N), jnp.float32),
        pltpu.VMEM((RBLK, N), jnp.float32),
        pltpu.VMEM((RBLK, N), jnp.float32),
        pltpu.SemaphoreType.DMA,
        pltpu.SemaphoreType.DMA,
        pltpu.SemaphoreType.DMA,
        pltpu.SemaphoreType.DMA,
    ],
    compiler_params=pltpu.CompilerParams(needs_layout_passes=False),
)
def _masked_cumsum_sc(x_hbm, m_hbm, out_hbm, xv0, xv1, mv0, mv1, ov0, ov1,
                      sin0, sin1, sout0, sout1):
    wid = lax.axis_index("s") * NC + lax.axis_index("c")
    row0 = wid * ROWS_PER_W

    def blk_row(b):
        return row0 + jnp.minimum(b, NBLK - 1) * RBLK

    def start_in(b, xv, mv, sem):
        r = blk_row(b)
        pltpu.make_async_copy(x_hbm.at[pl.ds(r, RBLK)], xv, sem).start()
        pltpu.make_async_copy(m_hbm.at[pl.ds(r, RBLK)], mv, sem).start()

    def wait_in(xv, mv, sem):
        pltpu.make_async_copy(x_hbm.at[pl.ds(row0, RBLK)], xv, sem).wait()
        pltpu.make_async_copy(m_hbm.at[pl.ds(row0, RBLK)], mv, sem).wait()

    def start_out(b, ov, sem):
        r = blk_row(b)
        pltpu.make_async_copy(ov, out_hbm.at[pl.ds(r, RBLK)], sem).start()

    def wait_out(ov, sem):
        pltpu.make_async_copy(ov, out_hbm.at[pl.ds(row0, RBLK)], sem).wait()

    def compute_block(xv, mv, ov):
        def chunk(i, carries):
            sl = pl.ds(i * L, L)
            new = []
            for rr in range(RBLK):
                masked = xv[rr, sl] * mv[rr, sl]
                s = jnp.cumsum(masked)
                ov[rr, sl] = s + carries[rr]
                new.append(s[L - 1] + carries[rr])
            return tuple(new)

        lax.fori_loop(0, CHUNKS, chunk, (jnp.float32(0.0),) * RBLK)

    def do_pair(k, carry_unused):
        b0 = 2 * k
        b1 = 2 * k + 1
        wait_in(xv0, mv0, sin0)

        @pl.when(k > 0)
        def _():
            wait_out(ov0, sout0)

        compute_block(xv0, mv0, ov0)
        start_out(b0, ov0, sout0)
        start_in(b0 + 2, xv0, mv0, sin0)
        wait_in(xv1, mv1, sin1)

        @pl.when(k > 0)
        def _():
            wait_out(ov1, sout1)

        compute_block(xv1, mv1, ov1)
        start_out(b1, ov1, sout1)
        start_in(b1 + 2, xv1, mv1, sin1)
        return carry_unused

    start_in(0, xv0, mv0, sin0)
    start_in(1, xv1, mv1, sin1)
    lax.fori_loop(0, NBLK // 2, do_pair, 0)
    wait_out(ov0, sout0)
    wait_out(ov1, sout1)
    wait_in(xv0, mv0, sin0)
    wait_in(xv1, mv1, sin1)


def kernel(x, mask):
    mf = jnp.where(mask, jnp.float32(1.0), jnp.float32(0.0))
    return _masked_cumsum_sc(x, mf)


# speedup vs baseline: 4.3954x; 1.0017x over previous
import functools

import jax
import jax.numpy as jnp
from jax import lax
from jax.experimental import pallas as pl
from jax.experimental.pallas import tpu as pltpu
from jax.experimental.pallas import tpu_sc as plsc

N = 4096
L = 16
NC = 2
NS = 16
NW = NC * NS
ROWS_PER_W = N // NW
RBLK = 4
NBLK = ROWS_PER_W // RBLK
CHUNKS = N // L

_mesh = plsc.VectorSubcoreMesh(core_axis_name="c", subcore_axis_name="s")


@functools.partial(
    pl.kernel,
    out_type=jax.ShapeDtypeStruct((N, N), jnp.float32),
    mesh=_mesh,
    scratch_types=[
        pltpu.VMEM((RBLK, N), jnp.float32),
        pltpu.VMEM((RBLK, N), jnp.float32),
        pltpu.VMEM((RBLK, N), jnp.float32),
        pltpu.VMEM((RBLK, N), jnp.float32),
        pltpu.VMEM((RBLK, N), jnp.float32),
        pltpu.VMEM((RBLK, N), jnp.float32),
        pltpu.SemaphoreType.DMA,
        pltpu.SemaphoreType.DMA,
        pltpu.SemaphoreType.DMA,
        pltpu.SemaphoreType.DMA,
    ],
    compiler_params=pltpu.CompilerParams(needs_layout_passes=False),
)
def _masked_cumsum_sc(x_hbm, m_hbm, out_hbm, xv0, xv1, mv0, mv1, ov0, ov1,
                      sin0, sin1, sout0, sout1):
    wid = lax.axis_index("s") * NC + lax.axis_index("c")
    row0 = wid * ROWS_PER_W

    def blk_row(b):
        return row0 + jnp.minimum(b, NBLK - 1) * RBLK

    def start_in(b, xv, mv, sem):
        r = blk_row(b)
        pltpu.make_async_copy(x_hbm.at[pl.ds(r, RBLK)], xv, sem).start()
        pltpu.make_async_copy(m_hbm.at[pl.ds(r, RBLK)], mv, sem).start()

    def wait_in(xv, mv, sem):
        pltpu.make_async_copy(x_hbm.at[pl.ds(row0, RBLK)], xv, sem).wait()
        pltpu.make_async_copy(m_hbm.at[pl.ds(row0, RBLK)], mv, sem).wait()

    def start_out(b, ov, sem):
        r = blk_row(b)
        pltpu.make_async_copy(ov, out_hbm.at[pl.ds(r, RBLK)], sem).start()

    def wait_out(ov, sem):
        pltpu.make_async_copy(ov, out_hbm.at[pl.ds(row0, RBLK)], sem).wait()

    def compute_block(xv, mv, ov):
        def chunk(i, carries):
            sl = pl.ds(i * L, L)
            new = []
            for rr in range(RBLK):
                masked = xv[rr, sl] * mv[rr, sl]
                s = jnp.cumsum(masked)
                ov[rr, sl] = s + carries[rr]
                new.append(s[L - 1] + carries[rr])
            return tuple(new)

        lax.fori_loop(0, CHUNKS, chunk, (jnp.float32(0.0),) * RBLK)

    def do_pair(k, carry_unused):
        b0 = 2 * k
        b1 = 2 * k + 1
        wait_in(xv0, mv0, sin0)

        @pl.when(k > 0)
        def _():
            wait_out(ov0, sout0)

        compute_block(xv0, mv0, ov0)
        start_out(b0, ov0, sout0)
        start_in(b0 + 2, xv0, mv0, sin0)
        wait_in(xv1, mv1, sin1)

        @pl.when(k > 0)
        def _():
            wait_out(ov1, sout1)

        compute_block(xv1, mv1, ov1)
        start_out(b1, ov1, sout1)
        start_in(b1 + 2, xv1, mv1, sin1)
        return carry_unused

    start_in(0, xv0, mv0, sin0)
    start_in(1, xv1, mv1, sin1)
    lax.fori_loop(0, NBLK // 2, do_pair, 0)
    wait_out(ov0, sout0)
    wait_out(ov1, sout1)
    wait_in(xv0, mv0, sin0)
    wait_in(xv1, mv1, sin1)


def kernel(x, mask):
    return _masked_cumsum_sc(x, mask.astype(jnp.float32))
